# baseline (speedup 1.0000x reference)
import functools

import jax
import jax.numpy as jnp
from jax import lax
from jax.experimental import pallas as pl
from jax.experimental.pallas import tpu as pltpu
from jax.experimental.pallas import tpu_sc as plsc

_LANES = 16
_GATHER = 128


def _sc_gather(bbs_flat, table, T, V, D, R):
    N = R * T
    info = plsc.get_sparse_core_info()
    NW = info.num_cores * info.num_subcores
    per_w = N // NW
    CH = T * _GATHER
    RPC = _GATHER
    n_chunks = per_w // CH

    mesh = plsc.VectorSubcoreMesh(core_axis_name="c", subcore_axis_name="s")

    @functools.partial(
        pl.kernel,
        mesh=mesh,
        out_type=jax.ShapeDtypeStruct((T, R, D), jnp.float32),
        scratch_types=[
            pltpu.VMEM((2, CH), jnp.int32),
            pltpu.VMEM((2, T, _GATHER), jnp.int32),
            pltpu.VMEM((2, CH, D), jnp.float32),
            pltpu.SemaphoreType.DMA,
            pltpu.SemaphoreType.DMA,
            pltpu.SemaphoreType.DMA,
            pltpu.SemaphoreType.DMA,
        ],
    )
    def k(idx_hbm, w_hbm, out_hbm, idxraw, idxadj, rows, g0, g1, s0, s1):
        wid = lax.axis_index("s") * info.num_cores + lax.axis_index("c")
        base0 = wid * per_w
        rbase0 = wid * (per_w // T)
        iota = lax.iota(jnp.int32, _LANES)
        gsem = (g0, g1)
        ssem = (s0, s1)

        def load_adjust(c, b):
            pltpu.sync_copy(idx_hbm.at[pl.ds(base0 + c * CH, CH)], idxraw.at[b])
            for g in range(CH // _LANES):
                off = ((iota + _LANES * g) % T) * V
                v = idxraw[b, pl.ds(_LANES * g, _LANES)] + off
                p = _LANES * g
                idxadj[b, p // _GATHER, pl.ds(p % _GATHER, _LANES)] = v

        def fire_gathers(b):
            for j in range(T):
                pltpu.async_copy(
                    w_hbm.at[idxadj.at[b].at[j]],
                    rows.at[b].at[pl.ds(j * _GATHER, _GATHER)],
                    gsem[b],
                )

        def wait_gathers(b):
            for j in range(T):
                pltpu.make_async_copy(
                    w_hbm.at[idxadj.at[b].at[j]],
                    rows.at[b].at[pl.ds(j * _GATHER, _GATHER)],
                    gsem[b],
                ).wait()

        def chunk_op(c, b, prefetch):
            wait_gathers(b)
            rbase = rbase0 + c * RPC
            rows3 = rows.at[b].reshape(RPC, T, D)
            scs = [
                pltpu.async_copy(
                    rows3.at[:, t, :], out_hbm.at[t, pl.ds(rbase, RPC)], ssem[b]
                )
                for t in range(T)
            ]
            if prefetch:
                load_adjust(c + 2, b)
            for sc in scs:
                sc.wait()
            if prefetch:
                fire_gathers(b)

        for b in range(2):
            load_adjust(b, b)
            fire_gathers(b)

        def body(kk, carry):
            for b in range(2):
                chunk_op(2 * kk + b, b, True)
            return carry

        lax.fori_loop(0, n_chunks // 2 - 1, body, 0)
        for b in range(2):
            chunk_op(n_chunks - 2 + b, b, False)

    return k(bbs_flat, table)


def _tc_interleave(x, T, D, R, block_rows=1024):
    def body(x_ref, o_ref):
        for t in range(T):
            o_ref[:, t * D:(t + 1) * D] = x_ref[t]

    return pl.pallas_call(
        body,
        out_shape=jax.ShapeDtypeStruct((R, T * D), jnp.float32),
        grid=(R // block_rows,),
        in_specs=[
            pl.BlockSpec((T, block_rows, D), lambda i: (0, i, 0)),
        ],
        out_specs=pl.BlockSpec((block_rows, T * D), lambda i: (i, 0)),
    )(x)


def kernel(bbs_inf, phi_W, psi_W, omega_W):
    B, L, T = bbs_inf.shape
    V, D = phi_W.shape
    R = B * L

    table = jnp.concatenate([phi_W, psi_W, omega_W], axis=0)
    idx_flat = bbs_inf.reshape(R * T)

    per_table = _sc_gather(idx_flat, table, T, V, D, R)
    out = _tc_interleave(per_table, T, D, R)
    return out.reshape(B, L, T * D)


# speedup vs baseline: 2.1219x; 2.1219x over previous
import functools

import jax
import jax.numpy as jnp
from jax import lax
from jax.experimental import pallas as pl
from jax.experimental.pallas import tpu as pltpu
from jax.experimental.pallas import tpu_sc as plsc

_GATHER = 128


def _sc_gather(idxs, tables, T, D, R):
    info = plsc.get_sparse_core_info()
    NW = info.num_cores * info.num_subcores
    rows_per_w = R // NW
    n_chunks = rows_per_w // _GATHER

    mesh = plsc.VectorSubcoreMesh(core_axis_name="c", subcore_axis_name="s")

    @functools.partial(
        pl.kernel,
        mesh=mesh,
        out_type=jax.ShapeDtypeStruct((T, R, D), jnp.float32),
        scratch_types=[
            pltpu.VMEM((2, T, _GATHER), jnp.int32),
            pltpu.VMEM((2, T, _GATHER, D), jnp.float32),
            pltpu.SemaphoreType.DMA,
            pltpu.SemaphoreType.DMA,
            pltpu.SemaphoreType.DMA,
            pltpu.SemaphoreType.DMA,
        ],
    )
    def k(i0, i1, i2, w0, w1, w2, out_hbm, idxvm, rows, g0, g1, s0, s1):
        idx_hbm = (i0, i1, i2)
        w_hbm = (w0, w1, w2)
        wid = lax.axis_index("s") * info.num_cores + lax.axis_index("c")
        rbase0 = wid * rows_per_w
        gsem = (g0, g1)
        ssem = (s0, s1)

        def load_idx(c, b):
            rbase = rbase0 + c * _GATHER
            for t in range(T):
                pltpu.sync_copy(
                    idx_hbm[t].at[pl.ds(rbase, _GATHER)], idxvm.at[b].at[t]
                )

        def fire_gathers(b):
            for t in range(T):
                pltpu.async_copy(
                    w_hbm[t].at[idxvm.at[b].at[t]], rows.at[b].at[t], gsem[b]
                )

        def wait_gathers(b):
            for t in range(T):
                pltpu.make_async_copy(
                    w_hbm[t].at[idxvm.at[b].at[t]], rows.at[b].at[t], gsem[b]
                ).wait()

        def chunk_op(c, b, prefetch):
            wait_gathers(b)
            rbase = rbase0 + c * _GATHER
            scs = [
                pltpu.async_copy(
                    rows.at[b].at[t], out_hbm.at[t, pl.ds(rbase, _GATHER)],
                    ssem[b],
                )
                for t in range(T)
            ]
            if prefetch:
                load_idx(c + 2, b)
            for sc in scs:
                sc.wait()
            if prefetch:
                fire_gathers(b)

        for b in range(2):
            load_idx(b, b)
            fire_gathers(b)

        def body(kk, carry):
            for b in range(2):
                chunk_op(2 * kk + b, b, True)
            return carry

        lax.fori_loop(0, n_chunks // 2 - 1, body, 0)
        for b in range(2):
            chunk_op(n_chunks - 2 + b, b, False)

    return k(*idxs, *tables)


def _tc_interleave(x, T, D, R, block_rows=1024):
    def body(x_ref, o_ref):
        for t in range(T):
            o_ref[:, t * D:(t + 1) * D] = x_ref[t]

    return pl.pallas_call(
        body,
        out_shape=jax.ShapeDtypeStruct((R, T * D), jnp.float32),
        grid=(R // block_rows,),
        in_specs=[
            pl.BlockSpec((T, block_rows, D), lambda i: (0, i, 0)),
        ],
        out_specs=pl.BlockSpec((block_rows, T * D), lambda i: (i, 0)),
    )(x)


def kernel(bbs_inf, phi_W, psi_W, omega_W):
    B, L, T = bbs_inf.shape
    V, D = phi_W.shape
    R = B * L

    idxs = [bbs_inf[:, :, t].reshape(R) for t in range(T)]
    per_table = _sc_gather(idxs, (phi_W, psi_W, omega_W), T, D, R)
    out = _tc_interleave(per_table, T, D, R)
    return out.reshape(B, L, T * D)
